# Initial kernel scaffold; baseline (speedup 1.0000x reference)
import jax
import jax.numpy as jnp
from jax.experimental import pallas as pl


def kernel(x_wave, x_transition, x_target, edge_index_wt, edge_index_tt, Wk_wave, bk_wave, Wv_wave, bv_wave, Wq_trans, bq_trans, Wk_trans, bk_trans, Wv_trans, bv_trans, Wq_tgt, bq_tgt, Wo_trans, bo_trans, Wo_tgt, bo_tgt, We_wt, be_wt, We_tt, be_tt, ln_w_wave, ln_b_wave, ln_w_trans, ln_b_trans, ln_w_tgt, ln_b_tgt):
    raise NotImplementedError("write your pallas kernel here")


# speedup vs baseline: 1.0368x; 1.0368x over previous
import functools

import jax
import jax.numpy as jnp
from jax.experimental import pallas as pl

N = 10000
D = 512
H = 8
DK = D // H
BLOCK_M = 400


def _proj_body(x_ref, w_ref, b_ref, o_ref):
    o_ref[...] = (
        jnp.dot(x_ref[...], w_ref[...], preferred_element_type=jnp.float32)
        + b_ref[...]
    )


def _fused_proj(x, w, b):
    wtot = w.shape[1]
    return pl.pallas_call(
        _proj_body,
        grid=(N // BLOCK_M,),
        in_specs=[
            pl.BlockSpec((BLOCK_M, D), lambda i: (i, 0)),
            pl.BlockSpec((D, wtot), lambda i: (0, 0)),
            pl.BlockSpec((1, wtot), lambda i: (0, 0)),
        ],
        out_specs=pl.BlockSpec((BLOCK_M, wtot), lambda i: (i, 0)),
        out_shape=jax.ShapeDtypeStruct((N, wtot), jnp.float32),
    )(x, w, b)


def _out_ln_body(msg_ref, wo_ref, bo_ref, x_ref, lw_ref, lb_ref, o_ref):
    h = (
        jnp.dot(msg_ref[...], wo_ref[...], preferred_element_type=jnp.float32)
        + bo_ref[...]
        + x_ref[...]
    )
    mu = jnp.mean(h, axis=-1, keepdims=True)
    var = jnp.mean((h - mu) ** 2, axis=-1, keepdims=True)
    o_ref[...] = (h - mu) * jax.lax.rsqrt(var + 1e-5) * lw_ref[...] + lb_ref[...]


def _fused_out_ln(msg, wo, bo, x, lw, lb):
    return pl.pallas_call(
        _out_ln_body,
        grid=(N // BLOCK_M,),
        in_specs=[
            pl.BlockSpec((BLOCK_M, D), lambda i: (i, 0)),
            pl.BlockSpec((D, D), lambda i: (0, 0)),
            pl.BlockSpec((1, D), lambda i: (0, 0)),
            pl.BlockSpec((BLOCK_M, D), lambda i: (i, 0)),
            pl.BlockSpec((1, D), lambda i: (0, 0)),
            pl.BlockSpec((1, D), lambda i: (0, 0)),
        ],
        out_specs=pl.BlockSpec((BLOCK_M, D), lambda i: (i, 0)),
        out_shape=jax.ShapeDtypeStruct((N, D), jnp.float32),
    )(msg, wo, bo, x, lw, lb)


def _ln_body(x_ref, lw_ref, lb_ref, o_ref):
    h = x_ref[...]
    mu = jnp.mean(h, axis=-1, keepdims=True)
    var = jnp.mean((h - mu) ** 2, axis=-1, keepdims=True)
    o_ref[...] = (h - mu) * jax.lax.rsqrt(var + 1e-5) * lw_ref[...] + lb_ref[...]


def _ln_only(x, lw, lb):
    return pl.pallas_call(
        _ln_body,
        grid=(N // BLOCK_M,),
        in_specs=[
            pl.BlockSpec((BLOCK_M, D), lambda i: (i, 0)),
            pl.BlockSpec((1, D), lambda i: (0, 0)),
            pl.BlockSpec((1, D), lambda i: (0, 0)),
        ],
        out_specs=pl.BlockSpec((BLOCK_M, D), lambda i: (i, 0)),
        out_shape=jax.ShapeDtypeStruct((N, D), jnp.float32),
    )(x, lw, lb)


def _edge_attention(q, kv, es, et, be, edge_index):
    tidx = edge_index[0]
    sidx = edge_index[1]
    eq = q[tidx].reshape(-1, H, DK)
    ekv = kv[sidx]
    ek = ekv[:, :D].reshape(-1, H, DK)
    ev = ekv[:, D:].reshape(-1, H, DK)
    scores = jnp.sum(eq * ek, axis=-1) / jnp.sqrt(float(DK))
    scores = scores + es[sidx] + et[tidx] + be
    m = jax.ops.segment_max(scores, tidx, num_segments=N)
    m = jnp.where(jnp.isfinite(m), m, 0.0)
    w = jnp.exp(scores - m[tidx])
    denom = jax.ops.segment_sum(w, tidx, num_segments=N)
    attn = w / denom[tidx]
    out = jax.ops.segment_sum(attn[..., None] * ev, tidx, num_segments=N)
    return out.reshape(N, D)


@jax.jit
def kernel(x_wave, x_transition, x_target, edge_index_wt, edge_index_tt,
           Wk_wave, bk_wave, Wv_wave, bv_wave, Wq_trans, bq_trans,
           Wk_trans, bk_trans, Wv_trans, bv_trans, Wq_tgt, bq_tgt,
           Wo_trans, bo_trans, Wo_tgt, bo_tgt, We_wt, be_wt, We_tt, be_tt,
           ln_w_wave, ln_b_wave, ln_w_trans, ln_b_trans, ln_w_tgt, ln_b_tgt):
    zpad = jnp.zeros((D, 128), jnp.float32)

    w_wave = jnp.concatenate([Wk_wave, Wv_wave, We_wt[:D],
                              zpad[:, : 128 - H]], axis=1)
    b_wave = jnp.concatenate(
        [bk_wave, bv_wave, jnp.zeros((128,), jnp.float32)])[None, :]
    a_wave = _fused_proj(x_wave, w_wave, b_wave)
    kv_wave = a_wave[:, : 2 * D]
    es_wt = a_wave[:, 2 * D : 2 * D + H]

    w_trans = jnp.concatenate(
        [Wq_trans, Wk_trans, Wv_trans, We_wt[D:], We_tt[:D],
         zpad[:, : 128 - 2 * H]], axis=1)
    b_trans = jnp.concatenate(
        [bq_trans, bk_trans, bv_trans, jnp.zeros((128,), jnp.float32)])[None, :]
    a_trans = _fused_proj(x_transition, w_trans, b_trans)
    q_trans = a_trans[:, :D]
    kv_trans = a_trans[:, D : 3 * D]
    et_wt = a_trans[:, 3 * D : 3 * D + H]
    es_tt = a_trans[:, 3 * D + H : 3 * D + 2 * H]

    w_tgt = jnp.concatenate([Wq_tgt, We_tt[D:], zpad[:, : 128 - H]], axis=1)
    b_tgt = jnp.concatenate(
        [bq_tgt, jnp.zeros((128,), jnp.float32)])[None, :]
    a_tgt = _fused_proj(x_target, w_tgt, b_tgt)
    q_tgt = a_tgt[:, :D]
    et_tt = a_tgt[:, D : D + H]

    msg_trans = _edge_attention(q_trans, kv_wave, es_wt, et_wt, be_wt,
                                edge_index_wt)
    msg_tgt = _edge_attention(q_tgt, kv_trans, es_tt, et_tt, be_tt,
                              edge_index_tt)

    y_wave = _ln_only(x_wave, ln_w_wave[None, :], ln_b_wave[None, :])
    y_trans = _fused_out_ln(msg_trans, Wo_trans, bo_trans[None, :],
                            x_transition, ln_w_trans[None, :],
                            ln_b_trans[None, :])
    y_tgt = _fused_out_ln(msg_tgt, Wo_tgt, bo_tgt[None, :],
                          x_target, ln_w_tgt[None, :], ln_b_tgt[None, :])
    return (y_wave, y_trans, y_tgt)


# speedup vs baseline: 1.0586x; 1.0210x over previous
import functools

import jax
import jax.numpy as jnp
from jax.experimental import pallas as pl

N = 10000
D = 512
H = 8
DK = D // H
BLOCK_M = 400


def _proj_body(x_ref, w_ref, b_ref, o_ref):
    o_ref[...] = (
        jnp.dot(x_ref[...], w_ref[...], preferred_element_type=jnp.float32)
        + b_ref[...]
    )


def _fused_proj(x, w, b):
    wtot = w.shape[1]
    return pl.pallas_call(
        _proj_body,
        grid=(N // BLOCK_M,),
        in_specs=[
            pl.BlockSpec((BLOCK_M, D), lambda i: (i, 0)),
            pl.BlockSpec((D, wtot), lambda i: (0, 0)),
            pl.BlockSpec((1, wtot), lambda i: (0, 0)),
        ],
        out_specs=pl.BlockSpec((BLOCK_M, wtot), lambda i: (i, 0)),
        out_shape=jax.ShapeDtypeStruct((N, wtot), jnp.float32),
    )(x, w, b)


def _out_ln_body(msg_ref, wo_ref, bo_ref, x_ref, lw_ref, lb_ref, o_ref):
    h = (
        jnp.dot(msg_ref[...], wo_ref[...], preferred_element_type=jnp.float32)
        + bo_ref[...]
        + x_ref[...]
    )
    mu = jnp.mean(h, axis=-1, keepdims=True)
    var = jnp.mean((h - mu) ** 2, axis=-1, keepdims=True)
    o_ref[...] = (h - mu) * jax.lax.rsqrt(var + 1e-5) * lw_ref[...] + lb_ref[...]


def _fused_out_ln(msg, wo, bo, x, lw, lb):
    return pl.pallas_call(
        _out_ln_body,
        grid=(N // BLOCK_M,),
        in_specs=[
            pl.BlockSpec((BLOCK_M, D), lambda i: (i, 0)),
            pl.BlockSpec((D, D), lambda i: (0, 0)),
            pl.BlockSpec((1, D), lambda i: (0, 0)),
            pl.BlockSpec((BLOCK_M, D), lambda i: (i, 0)),
            pl.BlockSpec((1, D), lambda i: (0, 0)),
            pl.BlockSpec((1, D), lambda i: (0, 0)),
        ],
        out_specs=pl.BlockSpec((BLOCK_M, D), lambda i: (i, 0)),
        out_shape=jax.ShapeDtypeStruct((N, D), jnp.float32),
    )(msg, wo, bo, x, lw, lb)


def _ln_body(x_ref, lw_ref, lb_ref, o_ref):
    h = x_ref[...]
    mu = jnp.mean(h, axis=-1, keepdims=True)
    var = jnp.mean((h - mu) ** 2, axis=-1, keepdims=True)
    o_ref[...] = (h - mu) * jax.lax.rsqrt(var + 1e-5) * lw_ref[...] + lb_ref[...]


def _ln_only(x, lw, lb):
    return pl.pallas_call(
        _ln_body,
        grid=(N // BLOCK_M,),
        in_specs=[
            pl.BlockSpec((BLOCK_M, D), lambda i: (i, 0)),
            pl.BlockSpec((1, D), lambda i: (0, 0)),
            pl.BlockSpec((1, D), lambda i: (0, 0)),
        ],
        out_specs=pl.BlockSpec((BLOCK_M, D), lambda i: (i, 0)),
        out_shape=jax.ShapeDtypeStruct((N, D), jnp.float32),
    )(x, lw, lb)


def _edge_attention(q, kv, es, et, be, edge_index):
    tidx = edge_index[0]
    sidx = edge_index[1]
    eq = q.astype(jnp.bfloat16)[tidx].reshape(-1, H, DK)
    ekv = kv.astype(jnp.bfloat16)[sidx]
    ek = ekv[:, :D].reshape(-1, H, DK)
    ev = ekv[:, D:].reshape(-1, H, DK).astype(jnp.float32)
    scores = jnp.sum(eq.astype(jnp.float32) * ek.astype(jnp.float32),
                     axis=-1) / jnp.sqrt(float(DK))
    scores = scores + es[sidx] + et[tidx] + be
    m = jax.ops.segment_max(scores, tidx, num_segments=N)
    m = jnp.where(jnp.isfinite(m), m, 0.0)
    w = jnp.exp(scores - m[tidx])
    denom = jax.ops.segment_sum(w, tidx, num_segments=N)
    attn = w / denom[tidx]
    out = jax.ops.segment_sum(attn[..., None] * ev, tidx, num_segments=N)
    return out.reshape(N, D)


@jax.jit
def kernel(x_wave, x_transition, x_target, edge_index_wt, edge_index_tt,
           Wk_wave, bk_wave, Wv_wave, bv_wave, Wq_trans, bq_trans,
           Wk_trans, bk_trans, Wv_trans, bv_trans, Wq_tgt, bq_tgt,
           Wo_trans, bo_trans, Wo_tgt, bo_tgt, We_wt, be_wt, We_tt, be_tt,
           ln_w_wave, ln_b_wave, ln_w_trans, ln_b_trans, ln_w_tgt, ln_b_tgt):
    zpad = jnp.zeros((D, 128), jnp.float32)

    w_wave = jnp.concatenate([Wk_wave, Wv_wave, We_wt[:D],
                              zpad[:, : 128 - H]], axis=1)
    b_wave = jnp.concatenate(
        [bk_wave, bv_wave, jnp.zeros((128,), jnp.float32)])[None, :]
    a_wave = _fused_proj(x_wave, w_wave, b_wave)
    kv_wave = a_wave[:, : 2 * D]
    es_wt = a_wave[:, 2 * D : 2 * D + H]

    w_trans = jnp.concatenate(
        [Wq_trans, Wk_trans, Wv_trans, We_wt[D:], We_tt[:D],
         zpad[:, : 128 - 2 * H]], axis=1)
    b_trans = jnp.concatenate(
        [bq_trans, bk_trans, bv_trans, jnp.zeros((128,), jnp.float32)])[None, :]
    a_trans = _fused_proj(x_transition, w_trans, b_trans)
    q_trans = a_trans[:, :D]
    kv_trans = a_trans[:, D : 3 * D]
    et_wt = a_trans[:, 3 * D : 3 * D + H]
    es_tt = a_trans[:, 3 * D + H : 3 * D + 2 * H]

    w_tgt = jnp.concatenate([Wq_tgt, We_tt[D:], zpad[:, : 128 - H]], axis=1)
    b_tgt = jnp.concatenate(
        [bq_tgt, jnp.zeros((128,), jnp.float32)])[None, :]
    a_tgt = _fused_proj(x_target, w_tgt, b_tgt)
    q_tgt = a_tgt[:, :D]
    et_tt = a_tgt[:, D : D + H]

    msg_trans = _edge_attention(q_trans, kv_wave, es_wt, et_wt, be_wt,
                                edge_index_wt)
    msg_tgt = _edge_attention(q_tgt, kv_trans, es_tt, et_tt, be_tt,
                              edge_index_tt)

    y_wave = _ln_only(x_wave, ln_w_wave[None, :], ln_b_wave[None, :])
    y_trans = _fused_out_ln(msg_trans, Wo_trans, bo_trans[None, :],
                            x_transition, ln_w_trans[None, :],
                            ln_b_trans[None, :])
    y_tgt = _fused_out_ln(msg_tgt, Wo_tgt, bo_tgt[None, :],
                          x_target, ln_w_tgt[None, :], ln_b_tgt[None, :])
    return (y_wave, y_trans, y_tgt)
